# baseline (speedup 1.0000x reference)
import functools

import jax
import jax.numpy as jnp
from jax import lax
from jax.experimental import pallas as pl
from jax.experimental.pallas import tpu as pltpu
from jax.experimental.pallas import tpu_sc as plsc

N_NODES = 10000
N_EDGES = 320000
D_FEAT = 128

NC = 2
NS = 16
LANES = 16

E_PER_CORE = N_EDGES // NC
E_PER_TILE = E_PER_CORE // NS
CHUNK = 128
N_CHUNKS = E_PER_TILE // CHUNK
TAIL = E_PER_TILE - N_CHUNKS * CHUNK
ROWS_PER_TILE = N_NODES // NS


NBUF = 3
PREFETCH = 2


def _sc_body(x_hbm, src_hbm, dst_hbm, w_hbm, out_hbm,
             acc, srcv, gbuf, wbuf, sidx, tsrc, tdst, gsems, ssems, csems):
    c = lax.axis_index("c")
    s = lax.axis_index("s")
    ebase = c * E_PER_CORE + s * E_PER_TILE

    zero16 = jnp.zeros((LANES,), jnp.float32)
    for e in range(CHUNK):
        for q in range(D_FEAT // LANES):
            gbuf[0, e, pl.ds(q * LANES, LANES)] = zero16
    row0 = s * ROWS_PER_TILE
    off = 0
    while off < ROWS_PER_TILE:
        cnt = min(CHUNK, ROWS_PER_TILE - off)
        pltpu.sync_copy(gbuf.at[0, pl.ds(0, cnt)],
                        acc.at[pl.ds(row0 + off, cnt)])
        off += cnt
    plsc.subcore_barrier()

    def issue_src(j, b):
        pltpu.async_copy(src_hbm.at[pl.ds(ebase + j * CHUNK, CHUNK)],
                         srcv.at[b], csems.at[b])

    def issue_chunk(j, b):
        pltpu.make_async_copy(src_hbm.at[pl.ds(0, CHUNK)],
                              srcv.at[b], csems.at[b]).wait()
        pltpu.async_copy(x_hbm.at[srcv.at[b]],
                         gbuf.at[b], gsems.at[b])
        pltpu.async_copy(w_hbm.at[pl.ds(ebase + j * CHUNK, CHUNK)],
                         wbuf.at[pl.ds(b * CHUNK, CHUNK)], gsems.at[b])
        pltpu.async_copy(dst_hbm.at[pl.ds(ebase + j * CHUNK, CHUNK)],
                         sidx.at[b], gsems.at[b])

    def wait_chunk(b):
        pltpu.make_async_copy(x_hbm.at[pl.ds(0, CHUNK)],
                              gbuf.at[b], gsems.at[b]).wait()
        pltpu.make_async_copy(w_hbm.at[pl.ds(0, CHUNK)],
                              wbuf.at[pl.ds(0, CHUNK)], gsems.at[b]).wait()
        pltpu.make_async_copy(dst_hbm.at[pl.ds(0, CHUNK)],
                              sidx.at[b], gsems.at[b]).wait()

    def wait_scatter(b):
        pltpu.make_async_copy(x_hbm.at[pl.ds(0, CHUNK)],
                              gbuf.at[b], ssems.at[b]).wait()

    for k in range(NBUF):
        issue_src(k, k)
    for k in range(PREFETCH):
        issue_chunk(k, k)

    def chunk_body(j, carry):
        b = lax.rem(j, NBUF)
        wait_chunk(b)
        @pl.when(j + NBUF < N_CHUNKS)
        def _():
            issue_src(j + NBUF, b)

        bg = lax.rem(j + PREFETCH, NBUF)

        @pl.when(j + PREFETCH < N_CHUNKS)
        def _():
            @pl.when(j >= NBUF - PREFETCH)
            def _():
                wait_scatter(bg)
            issue_chunk(j + PREFETCH, bg)

        for g in range(CHUNK // LANES):
            w16 = wbuf[pl.ds(b * CHUNK + g * LANES, LANES)]
            for e16 in range(LANES):
                wv = lax.gather(
                    w16, jnp.full((LANES, 1), e16, jnp.int32),
                    lax.GatherDimensionNumbers(offset_dims=(),
                                               collapsed_slice_dims=(0,),
                                               start_index_map=(0,)),
                    slice_sizes=(1,),
                    mode=lax.GatherScatterMode.PROMISE_IN_BOUNDS)
                e = g * LANES + e16
                for q in range(D_FEAT // LANES):
                    sl = pl.ds(q * LANES, LANES)
                    gbuf[b, e, sl] = gbuf[b, e, sl] * wv
        pltpu.async_copy(gbuf.at[b], acc.at[sidx.at[b]], ssems.at[b], add=True)
        return carry

    lax.fori_loop(0, N_CHUNKS, chunk_body, 0)
    for j in range(max(N_CHUNKS - NBUF, 0), N_CHUNKS):
        wait_scatter(j % NBUF)

    tbase = ebase + N_CHUNKS * CHUNK
    pltpu.sync_copy(src_hbm.at[pl.ds(tbase, TAIL)], tsrc)
    pltpu.sync_copy(dst_hbm.at[pl.ds(tbase, TAIL)], tdst)
    pltpu.sync_copy(w_hbm.at[pl.ds(tbase, TAIL)], wbuf.at[pl.ds(0, TAIL)])
    pltpu.async_copy(x_hbm.at[tsrc], gbuf.at[0, pl.ds(0, TAIL)],
                     gsems.at[0]).wait()
    wt = wbuf[pl.ds(0, LANES)]
    for e16 in range(TAIL):
        wv = lax.gather(
            wt, jnp.full((LANES, 1), e16, jnp.int32),
            lax.GatherDimensionNumbers(offset_dims=(),
                                       collapsed_slice_dims=(0,),
                                       start_index_map=(0,)),
            slice_sizes=(1,),
            mode=lax.GatherScatterMode.PROMISE_IN_BOUNDS)
        for q in range(D_FEAT // LANES):
            sl = pl.ds(q * LANES, LANES)
            gbuf[0, e16, sl] = gbuf[0, e16, sl] * wv
    pltpu.sync_copy(gbuf.at[0, pl.ds(0, TAIL)], acc.at[tdst], add=True)

    plsc.subcore_barrier()
    pltpu.sync_copy(acc.at[pl.ds(row0, ROWS_PER_TILE)],
                    out_hbm.at[c, pl.ds(row0, ROWS_PER_TILE)])


_sc_aggregate = pl.kernel(
    _sc_body,
    out_type=jax.ShapeDtypeStruct((NC, N_NODES, D_FEAT), jnp.float32),
    mesh=plsc.VectorSubcoreMesh(core_axis_name="c", subcore_axis_name="s",
                                num_cores=NC, num_subcores=NS),
    scratch_types=[
        pltpu.VMEM_SHARED((N_NODES, D_FEAT), jnp.float32),
        pltpu.VMEM((NBUF, CHUNK), jnp.int32),
        pltpu.VMEM((NBUF, CHUNK, D_FEAT), jnp.float32),
        pltpu.VMEM((NBUF * CHUNK,), jnp.float32),
        pltpu.VMEM((NBUF, CHUNK), jnp.int32),
        pltpu.VMEM((TAIL,), jnp.int32),
        pltpu.VMEM((TAIL,), jnp.int32),
        pltpu.SemaphoreType.DMA((NBUF,)),
        pltpu.SemaphoreType.DMA((NBUF,)),
        pltpu.SemaphoreType.DMA((NBUF,)),
    ],
    compiler_params=pltpu.CompilerParams(use_tc_tiling_on_sc=False,
                                         needs_layout_passes=False),
)


def _add_body(p_ref, o_ref):
    o_ref[...] = p_ref[0] + p_ref[1]


_ROWS_BLK = 1000


def _combine(parts):
    return pl.pallas_call(
        _add_body,
        grid=(N_NODES // _ROWS_BLK,),
        in_specs=[pl.BlockSpec((NC, _ROWS_BLK, D_FEAT), lambda i: (0, i, 0))],
        out_specs=pl.BlockSpec((_ROWS_BLK, D_FEAT), lambda i: (i, 0)),
        out_shape=jax.ShapeDtypeStruct((N_NODES, D_FEAT), jnp.float32),
    )(parts)


def kernel(x_high, edge_index, edge_weight):
    dst = edge_index[0].astype(jnp.int32)
    src = edge_index[1].astype(jnp.int32)
    parts = _sc_aggregate(x_high, src, dst, edge_weight)
    return _combine(parts)


# speedup vs baseline: 1.0281x; 1.0281x over previous
import functools

import jax
import jax.numpy as jnp
from jax import lax
from jax.experimental import pallas as pl
from jax.experimental.pallas import tpu as pltpu
from jax.experimental.pallas import tpu_sc as plsc

N_NODES = 10000
N_EDGES = 320000
D_FEAT = 128

NC = 2
NS = 16
LANES = 16

E_PER_CORE = N_EDGES // NC
E_PER_TILE = E_PER_CORE // NS
CHUNK = 80
N_CHUNKS = E_PER_TILE // CHUNK
ROWS_PER_TILE = N_NODES // NS


NBUF = 3
PREFETCH = 2


def _sc_body(x_hbm, src_hbm, dst_hbm, w_hbm, out_hbm,
             acc, src_v, gbuf, wbuf, sidx, gsems, ssems):
    c = lax.axis_index("c")
    s = lax.axis_index("s")
    ebase = c * E_PER_CORE + s * E_PER_TILE

    pltpu.sync_copy(src_hbm.at[pl.ds(ebase, E_PER_TILE)], src_v)

    zero16 = jnp.zeros((LANES,), jnp.float32)
    for e in range(CHUNK):
        for q in range(D_FEAT // LANES):
            gbuf[0, e, pl.ds(q * LANES, LANES)] = zero16
    row0 = s * ROWS_PER_TILE
    off = 0
    while off < ROWS_PER_TILE:
        cnt = min(CHUNK, ROWS_PER_TILE - off)
        pltpu.sync_copy(gbuf.at[0, pl.ds(0, cnt)],
                        acc.at[pl.ds(row0 + off, cnt)])
        off += cnt
    plsc.subcore_barrier()

    def issue_chunk(j, b):
        pltpu.async_copy(x_hbm.at[src_v.at[pl.ds(j * CHUNK, CHUNK)]],
                         gbuf.at[b], gsems.at[b])
        pltpu.async_copy(w_hbm.at[pl.ds(ebase + j * CHUNK, CHUNK)],
                         wbuf.at[pl.ds(b * CHUNK, CHUNK)], gsems.at[b])
        pltpu.async_copy(dst_hbm.at[pl.ds(ebase + j * CHUNK, CHUNK)],
                         sidx.at[b], gsems.at[b])

    def wait_chunk(b):
        pltpu.make_async_copy(x_hbm.at[pl.ds(0, CHUNK)],
                              gbuf.at[b], gsems.at[b]).wait()
        pltpu.make_async_copy(w_hbm.at[pl.ds(0, CHUNK)],
                              wbuf.at[pl.ds(0, CHUNK)], gsems.at[b]).wait()
        pltpu.make_async_copy(dst_hbm.at[pl.ds(0, CHUNK)],
                              sidx.at[b], gsems.at[b]).wait()

    def wait_scatter(b):
        pltpu.make_async_copy(x_hbm.at[pl.ds(0, CHUNK)],
                              gbuf.at[b], ssems.at[b]).wait()

    issue_chunk(0, 0)
    issue_chunk(1, 1)

    def chunk_body(j, carry):
        b = lax.rem(j, NBUF)
        wait_chunk(b)
        bg = lax.rem(j + PREFETCH, NBUF)

        @pl.when(j + PREFETCH < N_CHUNKS)
        def _():
            @pl.when(j >= NBUF - PREFETCH)
            def _():
                wait_scatter(bg)
            issue_chunk(j + PREFETCH, bg)

        for g in range(CHUNK // LANES):
            w16 = wbuf[pl.ds(b * CHUNK + g * LANES, LANES)]
            for e16 in range(LANES):
                wv = lax.gather(
                    w16, jnp.full((LANES, 1), e16, jnp.int32),
                    lax.GatherDimensionNumbers(offset_dims=(),
                                               collapsed_slice_dims=(0,),
                                               start_index_map=(0,)),
                    slice_sizes=(1,),
                    mode=lax.GatherScatterMode.PROMISE_IN_BOUNDS)
                e = g * LANES + e16
                for q in range(D_FEAT // LANES):
                    sl = pl.ds(q * LANES, LANES)
                    gbuf[b, e, sl] = gbuf[b, e, sl] * wv
        pltpu.async_copy(gbuf.at[b], acc.at[sidx.at[b]], ssems.at[b], add=True)
        return carry

    lax.fori_loop(0, N_CHUNKS, chunk_body, 0)
    for j in range(max(N_CHUNKS - NBUF, 0), N_CHUNKS):
        wait_scatter(j % NBUF)

    plsc.subcore_barrier()
    pltpu.sync_copy(acc.at[pl.ds(row0, ROWS_PER_TILE)],
                    out_hbm.at[c, pl.ds(row0, ROWS_PER_TILE)])


_sc_aggregate = pl.kernel(
    _sc_body,
    out_type=jax.ShapeDtypeStruct((NC, N_NODES, D_FEAT), jnp.float32),
    mesh=plsc.VectorSubcoreMesh(core_axis_name="c", subcore_axis_name="s",
                                num_cores=NC, num_subcores=NS),
    scratch_types=[
        pltpu.VMEM_SHARED((N_NODES, D_FEAT), jnp.float32),
        pltpu.VMEM((E_PER_TILE,), jnp.int32),
        pltpu.VMEM((NBUF, CHUNK, D_FEAT), jnp.float32),
        pltpu.VMEM((NBUF * CHUNK,), jnp.float32),
        pltpu.VMEM((NBUF, CHUNK), jnp.int32),
        pltpu.SemaphoreType.DMA((NBUF,)),
        pltpu.SemaphoreType.DMA((NBUF,)),
    ],
    compiler_params=pltpu.CompilerParams(use_tc_tiling_on_sc=False,
                                         needs_layout_passes=False),
)


def _add_body(p_ref, o_ref):
    o_ref[...] = p_ref[0] + p_ref[1]


_ROWS_BLK = 2000


def _combine(parts):
    return pl.pallas_call(
        _add_body,
        grid=(N_NODES // _ROWS_BLK,),
        in_specs=[pl.BlockSpec((NC, _ROWS_BLK, D_FEAT), lambda i: (0, i, 0))],
        out_specs=pl.BlockSpec((_ROWS_BLK, D_FEAT), lambda i: (i, 0)),
        out_shape=jax.ShapeDtypeStruct((N_NODES, D_FEAT), jnp.float32),
    )(parts)


def kernel(x_high, edge_index, edge_weight):
    dst = edge_index[0].astype(jnp.int32)
    src = edge_index[1].astype(jnp.int32)
    parts = _sc_aggregate(x_high, src, dst, edge_weight)
    return _combine(parts)
